# baseline (speedup 1.0000x reference)
import functools

import jax
import jax.numpy as jnp
from jax import lax
from jax.experimental import pallas as pl
from jax.experimental.pallas import tpu as pltpu
from jax.experimental.pallas import tpu_sc as plsc

_EMBED_DIM = 64
_BATCH = 4096
_HIST = 50
_NTOT = _BATCH * _HIST
_VOCAB = 1000000

_info = plsc.get_sparse_core_info()
_NC, _NS = _info.num_cores, _info.num_subcores
_NW = _NC * _NS

_mesh = plsc.VectorSubcoreMesh(core_axis_name="c", subcore_axis_name="s")

_GRP = 512
_VMAIN = 999424
_GPW = _VMAIN // _GRP // _NW
_NRING = 4
_STR = 16


@functools.partial(
    pl.kernel,
    mesh=_mesh,
    out_type=jax.ShapeDtypeStruct((_VOCAB * _EMBED_DIM,), jnp.float32),
    scratch_types=[
        pltpu.VMEM((2, _EMBED_DIM, _GRP), jnp.float32),
        pltpu.VMEM((_EMBED_DIM, 64), jnp.float32),
        pltpu.VMEM((_NRING * _STR * _EMBED_DIM,), jnp.float32),
        pltpu.SemaphoreType.DMA((2,)),
        pltpu.SemaphoreType.DMA((_NRING,)),
    ],
    compiler_params=pltpu.CompilerParams(
        use_tc_tiling_on_sc=True, needs_layout_passes=False
    ),
)
def _transpose_kernel(tt_hbm, tlin_hbm, buf_v, bufb_v, sbuf_v, gsem, ssem):
    wid = lax.axis_index("s") * _NC + lax.axis_index("c")
    iota64 = jnp.arange(_STR, dtype=jnp.int32) * _EMBED_DIM

    def load(g_local, bslot):
        v0 = pl.multiple_of((wid + _NW * g_local) * _GRP, _GRP)
        return pltpu.make_async_copy(
            tt_hbm.at[:, pl.ds(v0, _GRP)], buf_v.at[bslot], gsem.at[bslot]
        )

    def stripe(v0, vi, bufref, kslot, do_wait):
        kbase = kslot * _STR * _EMBED_DIM
        if do_wait:
            pltpu.make_async_copy(
                sbuf_v.at[pl.ds(kbase, _STR * _EMBED_DIM)],
                tlin_hbm.at[pl.ds(0, _STR * _EMBED_DIM)],
                ssem.at[kslot],
            ).wait()
        for d in range(_EMBED_DIM):
            vec = bufref[d, pl.ds(vi, _STR)]
            plsc.store_scatter(sbuf_v, [iota64 + (kbase + d)], vec)
        off = pl.multiple_of((v0 + vi) * _EMBED_DIM, 64)
        pltpu.make_async_copy(
            sbuf_v.at[pl.ds(kbase, _STR * _EMBED_DIM)],
            tlin_hbm.at[pl.ds(off, _STR * _EMBED_DIM)],
            ssem.at[kslot],
        ).start()

    def group_tail_stripes(v0, bslot, first_outer):
        @pl.loop(first_outer, _GRP // _STR // _NRING)
        def _go(so):
            for k in range(_NRING):
                stripe(v0, (so * _NRING + k) * _STR, buf_v.at[bslot], k, True)

    def v0_of(g_local):
        return pl.multiple_of((wid + _NW * g_local) * _GRP, _GRP)

    load(0, 0).start()
    load(1, 1).start()

    load(0, 0).wait()
    for k in range(_NRING):
        stripe(v0_of(0), k * _STR, buf_v.at[0], k, False)
    group_tail_stripes(v0_of(0), 0, 1)

    @pl.loop(0, (_GPW - 1) // 2)
    def _main(go):
        g1 = 1 + 2 * go
        load(g1, 1).wait()
        load(g1 + 1, 0).start()
        group_tail_stripes(v0_of(g1), 1, 0)
        g2 = 2 + 2 * go
        load(g2, 0).wait()

        @pl.when(go < (_GPW - 1) // 2 - 1)
        def _():
            load(g2 + 1, 1).start()

        group_tail_stripes(v0_of(g2), 0, 0)

    @pl.when(wid == 0)
    def _tail_a():
        pltpu.make_async_copy(
            tt_hbm.at[:, pl.ds(_VMAIN, _GRP)], buf_v.at[0], gsem.at[0]
        ).start()
        pltpu.make_async_copy(
            tt_hbm.at[:, pl.ds(_VMAIN, _GRP)], buf_v.at[0], gsem.at[0]
        ).wait()

        @pl.loop(0, _GRP // _STR // _NRING)
        def _ta(so):
            for k in range(_NRING):
                stripe(_VMAIN, (so * _NRING + k) * _STR, buf_v.at[0], k, True)

    @pl.when(wid == 1)
    def _tail_b():
        vb = _VMAIN + _GRP
        pltpu.make_async_copy(
            tt_hbm.at[:, pl.ds(vb, 64)], bufb_v, gsem.at[1]
        ).start()
        pltpu.make_async_copy(
            tt_hbm.at[:, pl.ds(vb, 64)], bufb_v, gsem.at[1]
        ).wait()
        for k in range(_NRING):
            stripe(vb, k * _STR, bufb_v, k, True)

    for k in range(_NRING):
        pltpu.make_async_copy(
            sbuf_v.at[pl.ds(k * _STR * _EMBED_DIM, _STR * _EMBED_DIM)],
            tlin_hbm.at[pl.ds(0, _STR * _EMBED_DIM)],
            ssem.at[k],
        ).wait()


_B_PER_W = _NTOT // _NW
_CHUNK = 256
_NCHUNK = _B_PER_W // _CHUNK
_NBUF = 5
_DEPTH = 2


@functools.partial(
    pl.kernel,
    mesh=_mesh,
    out_type=jax.ShapeDtypeStruct((_NTOT, _EMBED_DIM), jnp.float32),
    scratch_types=[
        pltpu.VMEM((_B_PER_W,), jnp.int32),
        pltpu.VMEM((_NBUF, _CHUNK, _EMBED_DIM), jnp.float32),
        pltpu.SemaphoreType.DMA((_NBUF,)),
        pltpu.SemaphoreType.DMA((_NBUF,)),
    ],
    compiler_params=pltpu.CompilerParams(use_tc_tiling_on_sc=False),
)
def _gather_kernel(table_hbm, idx_hbm, out_hbm, idx_v, rows_v, gsem, osem):
    wid = lax.axis_index("s") * _NC + lax.axis_index("c")
    base = wid * _B_PER_W
    pltpu.sync_copy(idx_hbm.at[pl.ds(base, _B_PER_W)], idx_v)

    def gather(g, b):
        off = pl.multiple_of(g * _CHUNK, 8)
        return pltpu.make_async_copy(
            table_hbm.at[idx_v.at[pl.ds(off, _CHUNK)]], rows_v.at[b], gsem.at[b]
        )

    def scatter(g, b):
        off = pl.multiple_of(g * _CHUNK, 8)
        return pltpu.make_async_copy(
            rows_v.at[b], out_hbm.at[pl.ds(base + off, _CHUNK)], osem.at[b]
        )

    for g in range(_DEPTH):
        gather(g, g).start()

    for g in range(_NBUF - _DEPTH):
        gather(g, g % _NBUF).wait()
        scatter(g, g % _NBUF).start()
        gather(g + _DEPTH, (g + _DEPTH) % _NBUF).start()

    _G0 = _NBUF - _DEPTH
    _NSTEADY = (_NCHUNK - _DEPTH) - _G0

    @pl.loop(0, _NSTEADY // _NBUF)
    def _steady(go):
        for db in range(_NBUF):
            g = _G0 + go * _NBUF + db
            b = (_G0 + db) % _NBUF
            gather(g, b).wait()
            scatter(g, b).start()
            f = g + _DEPTH
            bf = (_G0 + db + _DEPTH) % _NBUF
            scatter(f - _NBUF, bf).wait()
            gather(f, bf).start()

    for g in range(_NCHUNK - _DEPTH, _NCHUNK):
        b = g % _NBUF
        gather(g, b).wait()
        scatter(g, b).start()

    for g in range(_NCHUNK - _NBUF, _NCHUNK):
        scatter(g, g % _NBUF).wait()


def kernel(input_ids, table):
    tt = table.T
    tlin = _transpose_kernel(tt)
    t2 = tlin.reshape(_VOCAB, _EMBED_DIM)
    idx = input_ids.reshape(_NTOT).astype(jnp.int32)
    out = _gather_kernel(t2, idx)
    return out.reshape(_BATCH, _HIST, _EMBED_DIM)


# speedup vs baseline: 1.3466x; 1.3466x over previous
import functools

import jax
import jax.numpy as jnp
from jax import lax
from jax.experimental import pallas as pl
from jax.experimental.pallas import tpu as pltpu
from jax.experimental.pallas import tpu_sc as plsc

_EMBED_DIM = 64
_BATCH = 4096
_HIST = 50
_NTOT = _BATCH * _HIST
_VOCAB = 1000000

_info = plsc.get_sparse_core_info()
_NC, _NS = _info.num_cores, _info.num_subcores
_NW = _NC * _NS

_mesh = plsc.VectorSubcoreMesh(core_axis_name="c", subcore_axis_name="s")

_GRP = 512
_VMAIN = 999424
_GPW = _VMAIN // _GRP // _NW
_NRING = 4
_STR = 16


@functools.partial(
    pl.kernel,
    mesh=_mesh,
    out_type=jax.ShapeDtypeStruct((_VOCAB * _EMBED_DIM,), jnp.float32),
    scratch_types=[
        pltpu.VMEM((2, _EMBED_DIM, _GRP), jnp.float32),
        pltpu.VMEM((_EMBED_DIM, 64), jnp.float32),
        pltpu.VMEM((_NRING * _STR * _EMBED_DIM,), jnp.float32),
        pltpu.SemaphoreType.DMA((2,)),
        pltpu.SemaphoreType.DMA((_NRING,)),
    ],
    compiler_params=pltpu.CompilerParams(
        use_tc_tiling_on_sc=True, needs_layout_passes=False
    ),
)
def _transpose_kernel(tt_hbm, tlin_hbm, buf_v, bufb_v, sbuf_v, gsem, ssem):
    wid = lax.axis_index("s") * _NC + lax.axis_index("c")
    iota64 = jnp.arange(_STR, dtype=jnp.int32) * _EMBED_DIM

    def load(g_local, bslot):
        v0 = pl.multiple_of((wid + _NW * g_local) * _GRP, _GRP)
        return pltpu.make_async_copy(
            tt_hbm.at[:, pl.ds(v0, _GRP)], buf_v.at[bslot], gsem.at[bslot]
        )

    def stripe(v0, vi, bufref, kslot, do_wait):
        kbase = kslot * _STR * _EMBED_DIM
        if do_wait:
            pltpu.make_async_copy(
                sbuf_v.at[pl.ds(kbase, _STR * _EMBED_DIM)],
                tlin_hbm.at[pl.ds(0, _STR * _EMBED_DIM)],
                ssem.at[kslot],
            ).wait()
        @plsc.parallel_loop(0, _EMBED_DIM, unroll=16)
        def _pd(d):
            vec = bufref[d, pl.ds(vi, _STR)]
            plsc.store_scatter(sbuf_v, [iota64 + (kbase + d)], vec)
        off = pl.multiple_of((v0 + vi) * _EMBED_DIM, 64)
        pltpu.make_async_copy(
            sbuf_v.at[pl.ds(kbase, _STR * _EMBED_DIM)],
            tlin_hbm.at[pl.ds(off, _STR * _EMBED_DIM)],
            ssem.at[kslot],
        ).start()

    def group_tail_stripes(v0, bslot, first_outer):
        @pl.loop(first_outer, _GRP // _STR // _NRING)
        def _go(so):
            for k in range(_NRING):
                stripe(v0, (so * _NRING + k) * _STR, buf_v.at[bslot], k, True)

    def v0_of(g_local):
        return pl.multiple_of((wid + _NW * g_local) * _GRP, _GRP)

    load(0, 0).start()
    load(1, 1).start()

    load(0, 0).wait()
    for k in range(_NRING):
        stripe(v0_of(0), k * _STR, buf_v.at[0], k, False)
    group_tail_stripes(v0_of(0), 0, 1)

    @pl.loop(0, (_GPW - 1) // 2)
    def _main(go):
        g1 = 1 + 2 * go
        load(g1, 1).wait()
        load(g1 + 1, 0).start()
        group_tail_stripes(v0_of(g1), 1, 0)
        g2 = 2 + 2 * go
        load(g2, 0).wait()

        @pl.when(go < (_GPW - 1) // 2 - 1)
        def _():
            load(g2 + 1, 1).start()

        group_tail_stripes(v0_of(g2), 0, 0)

    @pl.when(wid == 0)
    def _tail_a():
        pltpu.make_async_copy(
            tt_hbm.at[:, pl.ds(_VMAIN, _GRP)], buf_v.at[0], gsem.at[0]
        ).start()
        pltpu.make_async_copy(
            tt_hbm.at[:, pl.ds(_VMAIN, _GRP)], buf_v.at[0], gsem.at[0]
        ).wait()

        @pl.loop(0, _GRP // _STR // _NRING)
        def _ta(so):
            for k in range(_NRING):
                stripe(_VMAIN, (so * _NRING + k) * _STR, buf_v.at[0], k, True)

    @pl.when(wid == 1)
    def _tail_b():
        vb = _VMAIN + _GRP
        pltpu.make_async_copy(
            tt_hbm.at[:, pl.ds(vb, 64)], bufb_v, gsem.at[1]
        ).start()
        pltpu.make_async_copy(
            tt_hbm.at[:, pl.ds(vb, 64)], bufb_v, gsem.at[1]
        ).wait()
        for k in range(_NRING):
            stripe(vb, k * _STR, bufb_v, k, True)

    for k in range(_NRING):
        pltpu.make_async_copy(
            sbuf_v.at[pl.ds(k * _STR * _EMBED_DIM, _STR * _EMBED_DIM)],
            tlin_hbm.at[pl.ds(0, _STR * _EMBED_DIM)],
            ssem.at[k],
        ).wait()


_B_PER_W = _NTOT // _NW
_CHUNK = 256
_NCHUNK = _B_PER_W // _CHUNK
_NBUF = 5
_DEPTH = 2


@functools.partial(
    pl.kernel,
    mesh=_mesh,
    out_type=jax.ShapeDtypeStruct((_NTOT, _EMBED_DIM), jnp.float32),
    scratch_types=[
        pltpu.VMEM((_B_PER_W,), jnp.int32),
        pltpu.VMEM((_NBUF, _CHUNK, _EMBED_DIM), jnp.float32),
        pltpu.SemaphoreType.DMA((_NBUF,)),
        pltpu.SemaphoreType.DMA((_NBUF,)),
    ],
    compiler_params=pltpu.CompilerParams(use_tc_tiling_on_sc=False),
)
def _gather_kernel(table_hbm, idx_hbm, out_hbm, idx_v, rows_v, gsem, osem):
    wid = lax.axis_index("s") * _NC + lax.axis_index("c")
    base = wid * _B_PER_W
    pltpu.sync_copy(idx_hbm.at[pl.ds(base, _B_PER_W)], idx_v)

    def gather(g, b):
        off = pl.multiple_of(g * _CHUNK, 8)
        return pltpu.make_async_copy(
            table_hbm.at[idx_v.at[pl.ds(off, _CHUNK)]], rows_v.at[b], gsem.at[b]
        )

    def scatter(g, b):
        off = pl.multiple_of(g * _CHUNK, 8)
        return pltpu.make_async_copy(
            rows_v.at[b], out_hbm.at[pl.ds(base + off, _CHUNK)], osem.at[b]
        )

    for g in range(_DEPTH):
        gather(g, g).start()

    for g in range(_NBUF - _DEPTH):
        gather(g, g % _NBUF).wait()
        scatter(g, g % _NBUF).start()
        gather(g + _DEPTH, (g + _DEPTH) % _NBUF).start()

    _G0 = _NBUF - _DEPTH
    _NSTEADY = (_NCHUNK - _DEPTH) - _G0

    @pl.loop(0, _NSTEADY // _NBUF)
    def _steady(go):
        for db in range(_NBUF):
            g = _G0 + go * _NBUF + db
            b = (_G0 + db) % _NBUF
            gather(g, b).wait()
            scatter(g, b).start()
            f = g + _DEPTH
            bf = (_G0 + db + _DEPTH) % _NBUF
            scatter(f - _NBUF, bf).wait()
            gather(f, bf).start()

    for g in range(_NCHUNK - _DEPTH, _NCHUNK):
        b = g % _NBUF
        gather(g, b).wait()
        scatter(g, b).start()

    for g in range(_NCHUNK - _NBUF, _NCHUNK):
        scatter(g, g % _NBUF).wait()


def kernel(input_ids, table):
    tt = table.T
    tlin = _transpose_kernel(tt)
    t2 = tlin.reshape(_VOCAB, _EMBED_DIM)
    idx = input_ids.reshape(_NTOT).astype(jnp.int32)
    out = _gather_kernel(t2, idx)
    return out.reshape(_BATCH, _HIST, _EMBED_DIM)
